# baseline (speedup 1.0000x reference)
import jax
import jax.numpy as jnp
from jax import lax
from jax.experimental import pallas as pl
from jax.experimental.pallas import tpu as pltpu
from jax.experimental.pallas import tpu_sc as plsc

_N = 10000
_E = 320000
_R = 16
_NB = 30
_D = 128
_NCORE = 2
_NSUB = 16
_NWORK = _NCORE * _NSUB
_CHUNK = 128
_TOTCH = _E // _CHUNK
_NPAD = 10240
_TROWS = _NPAD // _NSUB
_NR = _N * _R
_NR_T = _NR // _NSUB
_TN = 1000
_NT = _N // _TN

_NSUP = _TOTCH // _NWORK
_NMAIN = _NSUP * _NWORK
_NEXTRA = _TOTCH - _NMAIN
_CTILE = _TOTCH // _NSUB
_CEXTRA = _TOTCH - _CTILE * _NSUB


def _worker_id():
    return lax.axis_index("s") * _NCORE + lax.axis_index("c")


def _zero_rows(rows0):
    def _zrow(i, _):
        for b in range(_D // 16):
            rows0[i, pl.ds(b * 16, 16)] = jnp.zeros((16,), jnp.float32)
        return 0

    lax.fori_loop(0, _CHUNK, _zrow, 0)


def _zero_agg(rows0, agg, base):
    for m in range(_TROWS // _CHUNK):
        pltpu.sync_copy(rows0, agg.at[pl.ds(base + m * _CHUNK, _CHUNK)])


def _copy_out_agg(agg, rows0, out, c, base):
    for m in range(_TROWS // _CHUNK):
        pltpu.sync_copy(agg.at[pl.ds(base + m * _CHUNK, _CHUNK)], rows0)
        pltpu.sync_copy(rows0, out.at[c, pl.ds(base + m * _CHUNK, _CHUNK)])


def _run_main_loop(xw, packed, normsrc, agg, bufs, w):
    def _start_idx(p, k):
        j = w * _NSUP + k
        pltpu.async_copy(packed.at[pl.ds(8 * j, 8)], bufs["idx"][p],
                         bufs["isem"][p])

    def _wait_idx(p, k):
        j = w * _NSUP + k
        pltpu.make_async_copy(packed.at[pl.ds(8 * j, 8)], bufs["idx"][p],
                              bufs["isem"][p]).wait()

    def _start_gathers(p):
        pltpu.async_copy(xw.at[bufs["idx"][p].at[0]], bufs["rows"][p],
                         bufs["gsem"][p])
        pltpu.async_copy(normsrc.at[bufs["idx"][p].at[2]], bufs["norm"][p],
                         bufs["nsem"][p])

    def _wait_gathers(p):
        pltpu.make_async_copy(xw.at[bufs["idx"][p].at[0]], bufs["rows"][p],
                              bufs["gsem"][p]).wait()
        pltpu.make_async_copy(normsrc.at[bufs["idx"][p].at[2]],
                              bufs["norm"][p], bufs["nsem"][p]).wait()

    def _scale(p):
        rows_v, norm_v = bufs["rows"][p], bufs["norm"][p]

        def _g(g, _):
            nvec = norm_v[pl.ds(g * 16, 16)]
            for l in range(16):
                nv = nvec[l]
                e = g * 16 + l
                for b in range(_D // 16):
                    sl = pl.ds(b * 16, 16)
                    rows_v[e, sl] = rows_v[e, sl] * nv
            return 0

        lax.fori_loop(0, _CHUNK // 16, _g, 0)

    def _start_scatter(p):
        pltpu.async_copy(bufs["rows"][p], agg.at[bufs["idx"][p].at[1]],
                         bufs["ssem"][p], add=True)

    def _wait_scatter(p):
        pltpu.make_async_copy(bufs["rows"][p], agg.at[bufs["idx"][p].at[1]],
                              bufs["ssem"][p]).wait()

    _start_idx(0, 0)
    _wait_idx(0, 0)
    _start_gathers(0)

    def _process(cur, nxt, k):
        @pl.when(k >= 1)
        def _():
            _wait_scatter(nxt)

        @pl.when(k + 1 < _NSUP)
        def _():
            _start_idx(nxt, k + 1)

        _wait_gathers(cur)
        _scale(cur)

        @pl.when(k + 1 < _NSUP)
        def _():
            _wait_idx(nxt, k + 1)
            _start_gathers(nxt)

        _start_scatter(cur)

    def _step(k, _):
        @pl.when(k % 2 == 0)
        def _():
            _process(0, 1, k)

        @pl.when(k % 2 == 1)
        def _():
            _process(1, 0, k)

        return 0

    lax.fori_loop(0, _NSUP, _step, 0)
    _wait_scatter((_NSUP - 1) % 2)

    @pl.when(w < _NEXTRA)
    def _():
        j = _NMAIN + w
        pltpu.sync_copy(packed.at[pl.ds(8 * j, 8)], bufs["idx"][0])
        pltpu.async_copy(xw.at[bufs["idx"][0].at[0]], bufs["rows"][0],
                         bufs["gsem"][0]).wait()
        pltpu.async_copy(normsrc.at[bufs["idx"][0].at[2]], bufs["norm"][0],
                         bufs["nsem"][0]).wait()
        _scale(0)
        pltpu.sync_copy(bufs["rows"][0], agg.at[bufs["idx"][0].at[1]],
                        add=True)


def _edge1_body(xw, packed, pair2, out_agg, out_recip, bufs, agg, cnt_sh):
    c = lax.axis_index("c")
    s = lax.axis_index("s")
    w = _worker_id()
    rows0 = bufs["rows"][0]
    zbuf = bufs["zbuf"]
    ones_v = bufs["ones"]
    base = s * _TROWS
    cbase = s * _NR_T

    _zero_rows(rows0)
    _zero_agg(rows0, agg, base)

    def _zz(i, _):
        zbuf[pl.ds(i * 16, 16)] = jnp.zeros((16,), jnp.float32)
        return 0

    lax.fori_loop(0, 125, _zz, 0)
    for b in range(_CHUNK // 16):
        ones_v[pl.ds(b * 16, 16)] = jnp.ones((16,), jnp.float32)
    for m in range(5):
        pltpu.sync_copy(zbuf, cnt_sh.at[pl.ds(cbase + m * 2000, 2000)])
    plsc.subcore_barrier()

    ng = 19 + jnp.where(s < 8, 1, 0).astype(jnp.int32)
    gb = s * 19 + jnp.minimum(s, 8)

    def _cload(p, g):
        pltpu.sync_copy(pair2.at[pl.ds((gb + g) * 8, 8)], bufs["idx"][p])

    def _cstart(p):
        for r in range(8):
            pltpu.async_copy(ones_v, cnt_sh.at[bufs["idx"][p].at[r]],
                             bufs["gsem"][p], add=True)

    def _cwait(p):
        for r in range(8):
            pltpu.make_async_copy(ones_v, cnt_sh.at[bufs["idx"][p].at[r]],
                                  bufs["gsem"][p]).wait()

    _cload(0, 0)

    def _cproc(cur, nxt, g):
        @pl.when(g >= 1)
        def _():
            _cwait(nxt)

        @pl.when(g + 1 < ng)
        def _():
            _cload(nxt, g + 1)

        _cstart(cur)

    def _cstep(g, _):
        @pl.when(g % 2 == 0)
        def _():
            _cproc(0, 1, g)

        @pl.when(g % 2 == 1)
        def _():
            _cproc(1, 0, g)

        return 0

    lax.fori_loop(0, ng, _cstep, 0)

    @pl.when(ng % 2 == 1)
    def _():
        _cwait(0)

    @pl.when(ng % 2 == 0)
    def _():
        _cwait(1)

    @pl.when(s < _TOTCH - 312 * 8)
    def _():
        pltpu.sync_copy(pair2.at[pl.ds(312 * 8 + s, 1)],
                        bufs["idx"][0].at[pl.ds(0, 1)])
        pltpu.sync_copy(ones_v, cnt_sh.at[bufs["idx"][0].at[0]], add=True)

    plsc.subcore_barrier()

    for m in range(5):
        pltpu.sync_copy(cnt_sh.at[pl.ds(cbase + m * 2000, 2000)], zbuf)

        def _rr(i, _):
            sl = pl.ds(i * 16, 16)
            zbuf[sl] = 1.0 / jnp.maximum(zbuf[sl], 1.0)
            return 0

        lax.fori_loop(0, 125, _rr, 0)
        pltpu.sync_copy(zbuf, cnt_sh.at[pl.ds(cbase + m * 2000, 2000)])

        @pl.when(c == 0)
        def _():
            pltpu.sync_copy(zbuf, out_recip.at[pl.ds(cbase + m * 2000, 2000)])

    plsc.subcore_barrier()
    _run_main_loop(xw, packed, cnt_sh, agg, bufs, w)
    plsc.subcore_barrier()
    _copy_out_agg(agg, rows0, out_agg, c, base)


def _edge2_body(xw, packed, recip, out_agg, bufs, agg):
    c = lax.axis_index("c")
    s = lax.axis_index("s")
    w = _worker_id()
    rows0 = bufs["rows"][0]
    base = s * _TROWS
    _zero_rows(rows0)
    _zero_agg(rows0, agg, base)
    plsc.subcore_barrier()
    _run_main_loop(xw, packed, recip, agg, bufs, w)
    plsc.subcore_barrier()
    _copy_out_agg(agg, rows0, out_agg, c, base)


def _main_bufs():
    return dict(
        idx=[pltpu.VMEM((8, _CHUNK), jnp.int32) for _ in range(2)],
        norm=[pltpu.VMEM((_CHUNK,), jnp.float32) for _ in range(2)],
        rows=[pltpu.VMEM((_CHUNK, _D), jnp.float32) for _ in range(2)],
        gsem=[pltpu.SemaphoreType.DMA for _ in range(2)],
        isem=[pltpu.SemaphoreType.DMA for _ in range(2)],
        nsem=[pltpu.SemaphoreType.DMA for _ in range(2)],
        ssem=[pltpu.SemaphoreType.DMA for _ in range(2)],
    )


def _edge1_call(xw, packed, pair2):
    bufs = _main_bufs()
    bufs.update(
        ones=pltpu.VMEM((_CHUNK,), jnp.float32),
        zbuf=pltpu.VMEM((2000,), jnp.float32),
    )
    f = pl.kernel(
        _edge1_body,
        out_type=(jax.ShapeDtypeStruct((_NCORE, _NPAD, _D), jnp.float32),
                  jax.ShapeDtypeStruct((_NR,), jnp.float32)),
        mesh=plsc.VectorSubcoreMesh(core_axis_name="c", subcore_axis_name="s"),
        scratch_types=[
            bufs,
            pltpu.VMEM_SHARED((_NPAD, _D), jnp.float32),
            pltpu.VMEM_SHARED((_NR,), jnp.float32),
        ],
    )
    return f(xw, packed, pair2)


def _edge2_call(xw, packed, recip):
    f = pl.kernel(
        _edge2_body,
        out_type=jax.ShapeDtypeStruct((_NCORE, _NPAD, _D), jnp.float32),
        mesh=plsc.VectorSubcoreMesh(core_axis_name="c", subcore_axis_name="s"),
        scratch_types=[
            _main_bufs(),
            pltpu.VMEM_SHARED((_NPAD, _D), jnp.float32),
        ],
    )
    return f(xw, packed, recip)


def _eidx_body(src_ref, dst_ref, et_ref, out_ref, pair_ref):
    fidx = et_ref[...] * _N + src_ref[...]
    pair = dst_ref[...] * _R + et_ref[...]
    zeros = jnp.zeros((_TOTCH, 5, _CHUNK), jnp.int32)
    out_ref[...] = jnp.concatenate(
        [fidx[:, None, :], dst_ref[...][:, None, :], pair[:, None, :], zeros],
        axis=1)
    pair_ref[...] = pair


def _eidx_call(src2, dst2, et2):
    return pl.pallas_call(
        _eidx_body,
        out_shape=(jax.ShapeDtypeStruct((_TOTCH, 8, _CHUNK), jnp.int32),
                   jax.ShapeDtypeStruct((_TOTCH, _CHUNK), jnp.int32)),
    )(src2, dst2, et2)


def _xw_body(comp_ref, basis_ref, x_ref, o_ref):
    r = pl.program_id(0)
    onehot = (lax.broadcasted_iota(jnp.int32, (_R, _NB), 0) == r)
    cvec = jnp.sum(jnp.where(onehot, comp_ref[...], 0.0), axis=0)
    wmat = jnp.sum(cvec[:, None, None] * basis_ref[...], axis=0)
    o_ref[0] = jnp.dot(x_ref[...], wmat, preferred_element_type=jnp.float32)


def _xw_call(comp, basis, x):
    return pl.pallas_call(
        _xw_body,
        grid=(_R,),
        in_specs=[
            pl.BlockSpec((_R, _NB), lambda r: (0, 0)),
            pl.BlockSpec((_NB, _D, _D), lambda r: (0, 0, 0)),
            pl.BlockSpec((_N, _D), lambda r: (0, 0)),
        ],
        out_specs=pl.BlockSpec((1, _N, _D), lambda r: (r, 0, 0)),
        out_shape=jax.ShapeDtypeStruct((_R, _N, _D), jnp.float32),
    )(comp, basis, x)


def _combine_body(agg_ref, x_ref, root_ref, bias_ref, o_ref):
    acc = agg_ref[0] + agg_ref[1]
    acc = acc + jnp.dot(x_ref[...], root_ref[...],
                        preferred_element_type=jnp.float32)
    o_ref[...] = jnp.maximum(acc + bias_ref[...], 0.0)


def _combine_call(aggp, x, root, bias):
    return pl.pallas_call(
        _combine_body,
        grid=(_NT,),
        in_specs=[
            pl.BlockSpec((_NCORE, _TN, _D), lambda t: (0, t, 0)),
            pl.BlockSpec((_TN, _D), lambda t: (t, 0)),
            pl.BlockSpec((_D, _D), lambda t: (0, 0)),
            pl.BlockSpec((1, _D), lambda t: (0, 0)),
        ],
        out_specs=pl.BlockSpec((_TN, _D), lambda t: (t, 0)),
        out_shape=jax.ShapeDtypeStruct((_N, _D), jnp.float32),
    )(aggp, x, root, bias.reshape(1, _D))


def kernel(x, edge_index, edge_type, comp1, basis1, root1, bias1,
           comp2, basis2, root2, bias2):
    src2 = edge_index[0].reshape(_TOTCH, _CHUNK)
    dst2 = edge_index[1].reshape(_TOTCH, _CHUNK)
    et2 = edge_type.reshape(_TOTCH, _CHUNK)
    packed, pair2 = _eidx_call(src2, dst2, et2)
    packed = packed.reshape(8 * _TOTCH, _CHUNK)
    xw1 = _xw_call(comp1, basis1, x).reshape(_R * _N, _D)
    aggp1, recip = _edge1_call(xw1, packed, pair2)
    h = _combine_call(aggp1, x, root1, bias1)
    xw2 = _xw_call(comp2, basis2, h).reshape(_R * _N, _D)
    aggp2 = _edge2_call(xw2, packed, recip)
    return _combine_call(aggp2, h, root2, bias2)


# speedup vs baseline: 1.0439x; 1.0439x over previous
import jax
import jax.numpy as jnp
from jax import lax
from jax.experimental import pallas as pl
from jax.experimental.pallas import tpu as pltpu
from jax.experimental.pallas import tpu_sc as plsc

_N = 10000
_E = 320000
_R = 16
_NB = 30
_D = 128
_NCORE = 2
_NSUB = 16
_NWORK = _NCORE * _NSUB
_CHUNK = 128
_TOTCH = _E // _CHUNK
_NPAD = 10240
_TROWS = _NPAD // _NSUB
_NR = _N * _R
_NR_T = _NR // _NSUB
_TN = 1000
_NT = _N // _TN

_NSUP = _TOTCH // _NWORK
_NMAIN = _NSUP * _NWORK
_NEXTRA = _TOTCH - _NMAIN
_CTILE = _TOTCH // _NSUB
_CEXTRA = _TOTCH - _CTILE * _NSUB


def _worker_id():
    return lax.axis_index("s") * _NCORE + lax.axis_index("c")


def _zero_rows(rows0):
    def _zrow(i, _):
        for b in range(_D // 16):
            rows0[i, pl.ds(b * 16, 16)] = jnp.zeros((16,), jnp.float32)
        return 0

    lax.fori_loop(0, _CHUNK, _zrow, 0)


def _zero_agg(rows0, agg, base):
    for m in range(_TROWS // _CHUNK):
        pltpu.sync_copy(rows0, agg.at[pl.ds(base + m * _CHUNK, _CHUNK)])


def _copy_out_agg(agg, rows0, out, c, base):
    for m in range(_TROWS // _CHUNK):
        pltpu.sync_copy(agg.at[pl.ds(base + m * _CHUNK, _CHUNK)], rows0)
        pltpu.sync_copy(rows0, out.at[c, pl.ds(base + m * _CHUNK, _CHUNK)])


def _run_main_loop(xw, packed, normsrc, agg, bufs, w):
    def _start_idx(p, k):
        j = w * _NSUP + k
        pltpu.async_copy(packed.at[pl.ds(8 * j, 8)], bufs["idx"][p],
                         bufs["isem"][p])

    def _wait_idx(p, k):
        j = w * _NSUP + k
        pltpu.make_async_copy(packed.at[pl.ds(8 * j, 8)], bufs["idx"][p],
                              bufs["isem"][p]).wait()

    def _start_gathers(p):
        pltpu.async_copy(xw.at[bufs["idx"][p].at[0]], bufs["rows"][p],
                         bufs["gsem"][p])
        pltpu.async_copy(normsrc.at[bufs["idx"][p].at[2]], bufs["norm"][p],
                         bufs["nsem"][p])

    def _wait_gathers(p):
        pltpu.make_async_copy(xw.at[bufs["idx"][p].at[0]], bufs["rows"][p],
                              bufs["gsem"][p]).wait()
        pltpu.make_async_copy(normsrc.at[bufs["idx"][p].at[2]],
                              bufs["norm"][p], bufs["nsem"][p]).wait()

    def _scale(p):
        rows_v, norm_v = bufs["rows"][p], bufs["norm"][p]

        def _g(g, _):
            nvec = norm_v[pl.ds(g * 16, 16)]
            for l in range(16):
                nv = nvec[l]
                e = g * 16 + l
                for b in range(_D // 16):
                    sl = pl.ds(b * 16, 16)
                    rows_v[e, sl] = rows_v[e, sl] * nv
            return 0

        lax.fori_loop(0, _CHUNK // 16, _g, 0)

    def _start_scatter(p):
        pltpu.async_copy(bufs["rows"][p], agg.at[bufs["idx"][p].at[1]],
                         bufs["ssem"][p], add=True)

    def _wait_scatter(p):
        pltpu.make_async_copy(bufs["rows"][p], agg.at[bufs["idx"][p].at[1]],
                              bufs["ssem"][p]).wait()

    _start_idx(0, 0)
    _wait_idx(0, 0)
    _start_gathers(0)

    def _process(cur, nxt, k):
        @pl.when(k >= 1)
        def _():
            _wait_scatter(nxt)

        @pl.when(k + 1 < _NSUP)
        def _():
            _start_idx(nxt, k + 1)
            _wait_idx(nxt, k + 1)
            _start_gathers(nxt)

        _wait_gathers(cur)
        _scale(cur)
        _start_scatter(cur)

    def _step(k, _):
        @pl.when(k % 2 == 0)
        def _():
            _process(0, 1, k)

        @pl.when(k % 2 == 1)
        def _():
            _process(1, 0, k)

        return 0

    lax.fori_loop(0, _NSUP, _step, 0)
    _wait_scatter((_NSUP - 1) % 2)

    @pl.when(w < _NEXTRA)
    def _():
        j = _NMAIN + w
        pltpu.sync_copy(packed.at[pl.ds(8 * j, 8)], bufs["idx"][0])
        pltpu.async_copy(xw.at[bufs["idx"][0].at[0]], bufs["rows"][0],
                         bufs["gsem"][0]).wait()
        pltpu.async_copy(normsrc.at[bufs["idx"][0].at[2]], bufs["norm"][0],
                         bufs["nsem"][0]).wait()
        _scale(0)
        pltpu.sync_copy(bufs["rows"][0], agg.at[bufs["idx"][0].at[1]],
                        add=True)


def _edge1_body(xw, packed, pair2, out_agg, out_recip, bufs, agg, cnt_sh):
    c = lax.axis_index("c")
    s = lax.axis_index("s")
    w = _worker_id()
    rows0 = bufs["rows"][0]
    zbuf = bufs["zbuf"]
    ones_v = bufs["ones"]
    base = s * _TROWS
    cbase = s * _NR_T

    _zero_rows(rows0)
    _zero_agg(rows0, agg, base)

    def _zz(i, _):
        zbuf[pl.ds(i * 16, 16)] = jnp.zeros((16,), jnp.float32)
        return 0

    lax.fori_loop(0, 125, _zz, 0)
    for b in range(_CHUNK // 16):
        ones_v[pl.ds(b * 16, 16)] = jnp.ones((16,), jnp.float32)
    for m in range(5):
        pltpu.sync_copy(zbuf, cnt_sh.at[pl.ds(cbase + m * 2000, 2000)])
    plsc.subcore_barrier()

    ng = 19 + jnp.where(s < 8, 1, 0).astype(jnp.int32)
    gb = s * 19 + jnp.minimum(s, 8)

    def _cload(p, g):
        pltpu.sync_copy(pair2.at[pl.ds((gb + g) * 8, 8)], bufs["idx"][p])

    def _cstart(p):
        for r in range(8):
            pltpu.async_copy(ones_v, cnt_sh.at[bufs["idx"][p].at[r]],
                             bufs["gsem"][p], add=True)

    def _cwait(p):
        for r in range(8):
            pltpu.make_async_copy(ones_v, cnt_sh.at[bufs["idx"][p].at[r]],
                                  bufs["gsem"][p]).wait()

    _cload(0, 0)

    def _cproc(cur, nxt, g):
        @pl.when(g >= 1)
        def _():
            _cwait(nxt)

        @pl.when(g + 1 < ng)
        def _():
            _cload(nxt, g + 1)

        _cstart(cur)

    def _cstep(g, _):
        @pl.when(g % 2 == 0)
        def _():
            _cproc(0, 1, g)

        @pl.when(g % 2 == 1)
        def _():
            _cproc(1, 0, g)

        return 0

    lax.fori_loop(0, ng, _cstep, 0)

    @pl.when(ng % 2 == 1)
    def _():
        _cwait(0)

    @pl.when(ng % 2 == 0)
    def _():
        _cwait(1)

    @pl.when(s < _TOTCH - 312 * 8)
    def _():
        pltpu.sync_copy(pair2.at[pl.ds(312 * 8 + s, 1)],
                        bufs["idx"][0].at[pl.ds(0, 1)])
        pltpu.sync_copy(ones_v, cnt_sh.at[bufs["idx"][0].at[0]], add=True)

    plsc.subcore_barrier()

    for m in range(5):
        pltpu.sync_copy(cnt_sh.at[pl.ds(cbase + m * 2000, 2000)], zbuf)

        def _rr(i, _):
            sl = pl.ds(i * 16, 16)
            zbuf[sl] = 1.0 / jnp.maximum(zbuf[sl], 1.0)
            return 0

        lax.fori_loop(0, 125, _rr, 0)
        pltpu.sync_copy(zbuf, cnt_sh.at[pl.ds(cbase + m * 2000, 2000)])

        @pl.when(c == 0)
        def _():
            pltpu.sync_copy(zbuf, out_recip.at[pl.ds(cbase + m * 2000, 2000)])

    plsc.subcore_barrier()
    _run_main_loop(xw, packed, cnt_sh, agg, bufs, w)
    plsc.subcore_barrier()
    _copy_out_agg(agg, rows0, out_agg, c, base)


def _edge2_body(xw, packed, recip, out_agg, bufs, agg):
    c = lax.axis_index("c")
    s = lax.axis_index("s")
    w = _worker_id()
    rows0 = bufs["rows"][0]
    base = s * _TROWS
    _zero_rows(rows0)
    _zero_agg(rows0, agg, base)
    plsc.subcore_barrier()
    _run_main_loop(xw, packed, recip, agg, bufs, w)
    plsc.subcore_barrier()
    _copy_out_agg(agg, rows0, out_agg, c, base)


def _main_bufs():
    return dict(
        idx=[pltpu.VMEM((8, _CHUNK), jnp.int32) for _ in range(2)],
        norm=[pltpu.VMEM((_CHUNK,), jnp.float32) for _ in range(2)],
        rows=[pltpu.VMEM((_CHUNK, _D), jnp.float32) for _ in range(2)],
        gsem=[pltpu.SemaphoreType.DMA for _ in range(2)],
        isem=[pltpu.SemaphoreType.DMA for _ in range(2)],
        nsem=[pltpu.SemaphoreType.DMA for _ in range(2)],
        ssem=[pltpu.SemaphoreType.DMA for _ in range(2)],
    )


def _edge1_call(xw, packed, pair2):
    bufs = _main_bufs()
    bufs.update(
        ones=pltpu.VMEM((_CHUNK,), jnp.float32),
        zbuf=pltpu.VMEM((2000,), jnp.float32),
    )
    f = pl.kernel(
        _edge1_body,
        out_type=(jax.ShapeDtypeStruct((_NCORE, _NPAD, _D), jnp.float32),
                  jax.ShapeDtypeStruct((_NR,), jnp.float32)),
        mesh=plsc.VectorSubcoreMesh(core_axis_name="c", subcore_axis_name="s"),
        scratch_types=[
            bufs,
            pltpu.VMEM_SHARED((_NPAD, _D), jnp.float32),
            pltpu.VMEM_SHARED((_NR,), jnp.float32),
        ],
    )
    return f(xw, packed, pair2)


def _edge2_call(xw, packed, recip):
    f = pl.kernel(
        _edge2_body,
        out_type=jax.ShapeDtypeStruct((_NCORE, _NPAD, _D), jnp.float32),
        mesh=plsc.VectorSubcoreMesh(core_axis_name="c", subcore_axis_name="s"),
        scratch_types=[
            _main_bufs(),
            pltpu.VMEM_SHARED((_NPAD, _D), jnp.float32),
        ],
    )
    return f(xw, packed, recip)


def _eidx_body(src_ref, dst_ref, et_ref, out_ref, pair_ref):
    fidx = et_ref[...] * _N + src_ref[...]
    pair = dst_ref[...] * _R + et_ref[...]
    zeros = jnp.zeros((_TOTCH, 5, _CHUNK), jnp.int32)
    out_ref[...] = jnp.concatenate(
        [fidx[:, None, :], dst_ref[...][:, None, :], pair[:, None, :], zeros],
        axis=1)
    pair_ref[...] = pair


def _eidx_call(src2, dst2, et2):
    return pl.pallas_call(
        _eidx_body,
        out_shape=(jax.ShapeDtypeStruct((_TOTCH, 8, _CHUNK), jnp.int32),
                   jax.ShapeDtypeStruct((_TOTCH, _CHUNK), jnp.int32)),
    )(src2, dst2, et2)


def _xw_body(comp_ref, basis_ref, x_ref, o_ref):
    r = pl.program_id(0)
    onehot = (lax.broadcasted_iota(jnp.int32, (_R, _NB), 0) == r)
    cvec = jnp.sum(jnp.where(onehot, comp_ref[...], 0.0), axis=0)
    wmat = jnp.sum(cvec[:, None, None] * basis_ref[...], axis=0)
    o_ref[0] = jnp.dot(x_ref[...], wmat, preferred_element_type=jnp.float32)


def _xw_call(comp, basis, x):
    return pl.pallas_call(
        _xw_body,
        grid=(_R,),
        in_specs=[
            pl.BlockSpec((_R, _NB), lambda r: (0, 0)),
            pl.BlockSpec((_NB, _D, _D), lambda r: (0, 0, 0)),
            pl.BlockSpec((_N, _D), lambda r: (0, 0)),
        ],
        out_specs=pl.BlockSpec((1, _N, _D), lambda r: (r, 0, 0)),
        out_shape=jax.ShapeDtypeStruct((_R, _N, _D), jnp.float32),
    )(comp, basis, x)


def _combine_body(agg_ref, x_ref, root_ref, bias_ref, o_ref):
    acc = agg_ref[0] + agg_ref[1]
    acc = acc + jnp.dot(x_ref[...], root_ref[...],
                        preferred_element_type=jnp.float32)
    o_ref[...] = jnp.maximum(acc + bias_ref[...], 0.0)


def _combine_call(aggp, x, root, bias):
    return pl.pallas_call(
        _combine_body,
        grid=(_NT,),
        in_specs=[
            pl.BlockSpec((_NCORE, _TN, _D), lambda t: (0, t, 0)),
            pl.BlockSpec((_TN, _D), lambda t: (t, 0)),
            pl.BlockSpec((_D, _D), lambda t: (0, 0)),
            pl.BlockSpec((1, _D), lambda t: (0, 0)),
        ],
        out_specs=pl.BlockSpec((_TN, _D), lambda t: (t, 0)),
        out_shape=jax.ShapeDtypeStruct((_N, _D), jnp.float32),
    )(aggp, x, root, bias.reshape(1, _D))


def kernel(x, edge_index, edge_type, comp1, basis1, root1, bias1,
           comp2, basis2, root2, bias2):
    src2 = edge_index[0].reshape(_TOTCH, _CHUNK)
    dst2 = edge_index[1].reshape(_TOTCH, _CHUNK)
    et2 = edge_type.reshape(_TOTCH, _CHUNK)
    packed, pair2 = _eidx_call(src2, dst2, et2)
    packed = packed.reshape(8 * _TOTCH, _CHUNK)
    xw1 = _xw_call(comp1, basis1, x).reshape(_R * _N, _D)
    aggp1, recip = _edge1_call(xw1, packed, pair2)
    h = _combine_call(aggp1, x, root1, bias1)
    xw2 = _xw_call(comp2, basis2, h).reshape(_R * _N, _D)
    aggp2 = _edge2_call(xw2, packed, recip)
    return _combine_call(aggp2, h, root2, bias2)
